# baseline (speedup 1.0000x reference)
import functools

import jax
import jax.numpy as jnp
from jax import lax
from jax.experimental import pallas as pl
from jax.experimental.pallas import tpu as pltpu
from jax.experimental.pallas import tpu_sc as plsc

_NC = 2
_NS = 16
_C = 80


def _w_body(x_ref, wmat_ref, idx_ref, o_ref, seg_ref):
    er = jnp.dot(x_ref[...], wmat_ref[...], preferred_element_type=jnp.float32)
    e = jnp.where(er > 0, er, 0.01 * er)
    o_ref[...] = jnp.exp(e)
    seg_ref[...] = idx_ref[:, 0:1]


def kernel(feat_src, feat_dst, metapath_idx, attn_r):
    M, HD = feat_src.shape
    N = feat_dst.shape[0]
    H, D = attn_r.shape[1], attn_r.shape[2]
    HP = 16

    r_flat = attn_r.reshape(H * D).astype(jnp.float32)
    j = jnp.arange(HD)
    wmat = (j[:, None] // D == jnp.arange(HP)[None, :]) * r_flat[:, None]

    BM = 8000
    w_edges, seg = pl.pallas_call(
        _w_body,
        grid=(M // BM,),
        in_specs=[
            pl.BlockSpec((BM, HD), lambda i: (i, 0)),
            pl.BlockSpec((HD, HP), lambda i: (0, 0)),
            pl.BlockSpec((BM, 3), lambda i: (i, 0)),
        ],
        out_specs=[
            pl.BlockSpec((BM, HP), lambda i: (i, 0)),
            pl.BlockSpec((BM, 1), lambda i: (i, 0)),
        ],
        out_shape=[
            jax.ShapeDtypeStruct((M, HP), jnp.float32),
            jax.ShapeDtypeStruct((M, 1), jnp.int32),
        ],
    )(feat_src, wmat, metapath_idx)
    seg = seg.reshape(M)

    n_work = _NC * _NS
    m_per = M // n_work
    nchunk = m_per // _C
    nz = N // _C
    DR = N * H // HD

    mesh = plsc.VectorSubcoreMesh(core_axis_name="c", subcore_axis_name="s")

    @functools.partial(
        pl.kernel,
        out_type=(
            jax.ShapeDtypeStruct((N, HD), jnp.float32),
            jax.ShapeDtypeStruct((N, HD), jnp.float32),
        ),
        mesh=mesh,
        compiler_params=pltpu.CompilerParams(needs_layout_passes=False),
        scratch_types=[
            pltpu.VMEM((2, _C, HD), jnp.float32),
            pltpu.VMEM((2, _C, HP), jnp.float32),
            pltpu.VMEM((2, _C), jnp.int32),
            pltpu.VMEM_SHARED((N, HD), jnp.float32),
            pltpu.SemaphoreType.DMA((2,)),
            pltpu.SemaphoreType.DMA,
        ],
    )
    def sc_feats(x_hbm, w_hbm, seg_hbm, outf0_hbm, outf1_hbm, xbuf, wbuf,
                 idxbuf, acc, sem_in, sem_sc):
        c = lax.axis_index("c")
        s = lax.axis_index("s")
        wid = s * _NC + c
        zero = jnp.zeros((HP,), jnp.float32)
        base0 = wid * m_per

        def issue_loads(i, b):
            base = pl.multiple_of(base0 + i * _C, 8)
            pltpu.async_copy(x_hbm.at[pl.ds(base, _C)], xbuf.at[b],
                             sem_in.at[b])
            pltpu.async_copy(w_hbm.at[pl.ds(base, _C)], wbuf.at[b],
                             sem_in.at[b])
            pltpu.async_copy(seg_hbm.at[pl.ds(base, _C)], idxbuf.at[b],
                             sem_in.at[b])

        def wait_loads(b):
            pltpu.make_async_copy(x_hbm.at[pl.ds(0, _C)], xbuf.at[b],
                                  sem_in.at[b]).wait()
            pltpu.make_async_copy(w_hbm.at[pl.ds(0, _C)], wbuf.at[b],
                                  sem_in.at[b]).wait()
            pltpu.make_async_copy(seg_hbm.at[pl.ds(0, _C)], idxbuf.at[b],
                                  sem_in.at[b]).wait()

        def scale(b):
            @pl.loop(0, _C, unroll=2)
            def _(e):
                wv = wbuf[b, e, :]
                for h in range(H):
                    xbuf[b, e, h * D:(h + 1) * D] = (
                        xbuf[b, e, h * D:(h + 1) * D] * wv[h])

        def issue_scatter(b):
            pltpu.async_copy(xbuf.at[b], acc.at[idxbuf.at[b]], sem_sc,
                             add=True)

        def wait_scatter(b):
            pltpu.make_async_copy(xbuf.at[b], acc.at[idxbuf.at[b]],
                                  sem_sc).wait()

        @pl.loop(0, _C)
        def _(r):
            for k in range(HD // HP):
                xbuf[0, r, k * HP:(k + 1) * HP] = zero

        @pl.loop(s, nz, step=_NS)
        def _(q):
            off = pl.multiple_of(q * _C, 8)
            pltpu.sync_copy(xbuf.at[0], acc.at[pl.ds(off, _C)])
        plsc.subcore_barrier()

        issue_loads(0, 0)
        wait_loads(0)
        scale(0)
        issue_loads(1, 1)
        issue_scatter(0)

        @pl.loop(0, (nchunk - 1) // 2)
        def _(jj):
            i1 = 2 * jj + 1
            wait_loads(1)
            scale(1)
            wait_scatter(0)
            issue_loads(i1 + 1, 0)
            issue_scatter(1)

            i2 = 2 * jj + 2
            wait_loads(0)
            scale(0)
            wait_scatter(1)

            @pl.when(i2 + 1 < nchunk)
            def _():
                issue_loads(i2 + 1, 1)

            issue_scatter(0)

        wait_scatter((nchunk - 1) & 1)
        plsc.subcore_barrier()

        @pl.loop(s, nz, step=_NS)
        def _(q):
            off = pl.multiple_of(q * _C, 8)

            @pl.when(c == 0)
            def _():
                pltpu.sync_copy(acc.at[pl.ds(off, _C)], outf0_hbm.at[pl.ds(off, _C)])

            @pl.when(c == 1)
            def _():
                pltpu.sync_copy(acc.at[pl.ds(off, _C)], outf1_hbm.at[pl.ds(off, _C)])

    p0, p1 = sc_feats(feat_src, w_edges, seg)

    CD = _C
    nd = m_per // CD

    @functools.partial(
        pl.kernel,
        out_type=jax.ShapeDtypeStruct((n_work, N * H), jnp.float32),
        mesh=mesh,
        compiler_params=pltpu.CompilerParams(needs_layout_passes=False),
        scratch_types=[
            pltpu.VMEM((2, CD, HP), jnp.float32),
            pltpu.VMEM((2, CD), jnp.int32),
            pltpu.VMEM((N * H,), jnp.float32),
            pltpu.SemaphoreType.DMA((2,)),
        ],
    )
    def sc_denom(w_hbm, seg_hbm, outd_hbm, wbuf, idxbuf, dbuf, sem_in):
        c = lax.axis_index("c")
        s = lax.axis_index("s")
        wid = s * _NC + c
        zero = jnp.zeros((HP,), jnp.float32)
        iota = lax.iota(jnp.int32, HP)
        hmask = iota < H
        base0 = wid * m_per

        def issue_loads(i, b):
            base = pl.multiple_of(base0 + i * CD, 8)
            pltpu.async_copy(w_hbm.at[pl.ds(base, CD)], wbuf.at[b],
                             sem_in.at[b])
            pltpu.async_copy(seg_hbm.at[pl.ds(base, CD)], idxbuf.at[b],
                             sem_in.at[b])

        def wait_loads(b):
            pltpu.make_async_copy(w_hbm.at[pl.ds(0, CD)], wbuf.at[b],
                                  sem_in.at[b]).wait()
            pltpu.make_async_copy(seg_hbm.at[pl.ds(0, CD)], idxbuf.at[b],
                                  sem_in.at[b]).wait()

        issue_loads(0, 0)

        @pl.loop(0, N * H // HP)
        def _(r):
            dbuf[pl.ds(r * HP, HP)] = zero

        def den_accum(i, b):
            wait_loads(b)

            @pl.when(i + 1 < nd)
            def _():
                issue_loads(i + 1, 1 - b)

            @pl.loop(0, CD // HP)
            def _(g):
                iv = idxbuf[b, pl.ds(g * HP, HP)]
                for e16 in range(HP):
                    wv = wbuf[b, g * HP + e16, :]
                    flat = iv[e16] * H + iota
                    plsc.addupdate_scatter(dbuf, [flat], wv, mask=hmask)

        den_accum(0, 0)

        @pl.loop(0, (nd - 1) // 2)
        def _(jj):
            den_accum(2 * jj + 1, 1)
            den_accum(2 * jj + 2, 0)

        pltpu.sync_copy(dbuf, outd_hbm.at[wid])

    parts_d = sc_denom(w_edges, seg)

    NPR = HD // H
    FW = NPR * HD
    NR = N // NPR
    cols = jnp.arange(FW)
    expand = (jnp.arange(HD)[:, None]
              == (cols // HD) * H + (cols % HD) // D).astype(jnp.float32)

    p0v = p0.reshape(NR, FW)
    p1v = p1.reshape(NR, FW)
    den_parts = parts_d.reshape(n_work, DR, HD)

    def _combine_body(p0_ref, p1_ref, dp_ref, ex_ref, o_ref):
        u = p0_ref[...] + p1_ref[...]
        den = jnp.sum(dp_ref[...], axis=0)
        den_rep = jnp.dot(den, ex_ref[...],
                          preferred_element_type=jnp.float32)
        v = u / jnp.where(den_rep > 0, den_rep, 1.0)
        o_ref[...] = jnp.where(v > 0, v, jnp.exp(v) - 1.0)

    out = pl.pallas_call(
        _combine_body,
        grid=(1,),
        in_specs=[
            pl.BlockSpec((NR, FW), lambda i: (0, 0)),
            pl.BlockSpec((NR, FW), lambda i: (0, 0)),
            pl.BlockSpec((n_work, NR, HD), lambda i: (0, 0, 0)),
            pl.BlockSpec((HD, FW), lambda i: (0, 0)),
        ],
        out_specs=pl.BlockSpec((NR, FW), lambda i: (0, 0)),
        out_shape=jax.ShapeDtypeStruct((NR, FW), jnp.float32),
    )(p0v, p1v, den_parts, expand)
    return out.reshape(N, HD)


# speedup vs baseline: 1.3281x; 1.3281x over previous
import functools

import jax
import jax.numpy as jnp
from jax import lax
from jax.experimental import pallas as pl
from jax.experimental.pallas import tpu as pltpu
from jax.experimental.pallas import tpu_sc as plsc

_NC = 2
_NS = 16
_C = 80


def _w_body(x_ref, wmat_ref, o_ref):
    er = jnp.dot(x_ref[...], wmat_ref[...], preferred_element_type=jnp.float32)
    e = jnp.where(er > 0, er, 0.01 * er)
    o_ref[...] = jnp.exp(e)


def kernel(feat_src, feat_dst, metapath_idx, attn_r):
    M, HD = feat_src.shape
    N = feat_dst.shape[0]
    H, D = attn_r.shape[1], attn_r.shape[2]
    HP = 16

    r_flat = attn_r.reshape(H * D).astype(jnp.float32)
    j = jnp.arange(HD)
    wmat = (j[:, None] // D == jnp.arange(HP)[None, :]) * r_flat[:, None]

    BM = 8000
    seg = metapath_idx[:, 0]
    w_edges = pl.pallas_call(
        _w_body,
        grid=(M // BM,),
        in_specs=[
            pl.BlockSpec((BM, HD), lambda i: (i, 0)),
            pl.BlockSpec((HD, HP), lambda i: (0, 0)),
        ],
        out_specs=pl.BlockSpec((BM, HP), lambda i: (i, 0)),
        out_shape=jax.ShapeDtypeStruct((M, HP), jnp.float32),
    )(feat_src, wmat)

    n_work = _NC * _NS
    m_per = M // n_work
    nchunk = m_per // _C
    nz = N // _C
    DR = N * H // HD

    mesh = plsc.VectorSubcoreMesh(core_axis_name="c", subcore_axis_name="s")

    @functools.partial(
        pl.kernel,
        out_type=(
            jax.ShapeDtypeStruct((N, HD), jnp.float32),
            jax.ShapeDtypeStruct((N, HD), jnp.float32),
        ),
        mesh=mesh,
        compiler_params=pltpu.CompilerParams(needs_layout_passes=False),
        scratch_types=[
            pltpu.VMEM((2, _C, HD), jnp.float32),
            pltpu.VMEM((2, _C, HP), jnp.float32),
            pltpu.VMEM((2, _C), jnp.int32),
            pltpu.VMEM_SHARED((N, HD), jnp.float32),
            pltpu.SemaphoreType.DMA((2,)),
            pltpu.SemaphoreType.DMA,
        ],
    )
    def sc_feats(x_hbm, w_hbm, seg_hbm, outf0_hbm, outf1_hbm, xbuf, wbuf,
                 idxbuf, acc, sem_in, sem_sc):
        c = lax.axis_index("c")
        s = lax.axis_index("s")
        wid = s * _NC + c
        zero = jnp.zeros((HP,), jnp.float32)
        base0 = wid * m_per

        def issue_loads(i, b):
            base = pl.multiple_of(base0 + i * _C, 8)
            pltpu.async_copy(x_hbm.at[pl.ds(base, _C)], xbuf.at[b],
                             sem_in.at[b])
            pltpu.async_copy(w_hbm.at[pl.ds(base, _C)], wbuf.at[b],
                             sem_in.at[b])
            pltpu.async_copy(seg_hbm.at[pl.ds(base, _C)], idxbuf.at[b],
                             sem_in.at[b])

        def wait_loads(b):
            pltpu.make_async_copy(x_hbm.at[pl.ds(0, _C)], xbuf.at[b],
                                  sem_in.at[b]).wait()
            pltpu.make_async_copy(w_hbm.at[pl.ds(0, _C)], wbuf.at[b],
                                  sem_in.at[b]).wait()
            pltpu.make_async_copy(seg_hbm.at[pl.ds(0, _C)], idxbuf.at[b],
                                  sem_in.at[b]).wait()

        def scale(b):
            @pl.loop(0, _C, unroll=4)
            def _(e):
                wv = wbuf[b, e, :]
                for h in range(H):
                    xbuf[b, e, h * D:(h + 1) * D] = (
                        xbuf[b, e, h * D:(h + 1) * D] * wv[h])

        def issue_scatter(b):
            pltpu.async_copy(xbuf.at[b], acc.at[idxbuf.at[b]], sem_sc,
                             add=True)

        def wait_scatter(b):
            pltpu.make_async_copy(xbuf.at[b], acc.at[idxbuf.at[b]],
                                  sem_sc).wait()

        @pl.loop(0, _C)
        def _(r):
            for k in range(HD // HP):
                xbuf[0, r, k * HP:(k + 1) * HP] = zero

        @pl.loop(s, nz, step=_NS)
        def _(q):
            off = pl.multiple_of(q * _C, 8)
            pltpu.sync_copy(xbuf.at[0], acc.at[pl.ds(off, _C)])
        plsc.subcore_barrier()

        issue_loads(0, 0)
        wait_loads(0)
        scale(0)
        issue_loads(1, 1)
        issue_scatter(0)

        @pl.loop(0, (nchunk - 1) // 2)
        def _(jj):
            i1 = 2 * jj + 1
            wait_loads(1)
            scale(1)
            wait_scatter(0)
            issue_loads(i1 + 1, 0)
            issue_scatter(1)

            i2 = 2 * jj + 2
            wait_loads(0)
            scale(0)
            wait_scatter(1)

            @pl.when(i2 + 1 < nchunk)
            def _():
                issue_loads(i2 + 1, 1)

            issue_scatter(0)

        wait_scatter((nchunk - 1) & 1)
        plsc.subcore_barrier()

        @pl.loop(s, nz, step=_NS)
        def _(q):
            off = pl.multiple_of(q * _C, 8)

            @pl.when(c == 0)
            def _():
                pltpu.sync_copy(acc.at[pl.ds(off, _C)], outf0_hbm.at[pl.ds(off, _C)])

            @pl.when(c == 1)
            def _():
                pltpu.sync_copy(acc.at[pl.ds(off, _C)], outf1_hbm.at[pl.ds(off, _C)])

    p0, p1 = sc_feats(feat_src, w_edges, seg)

    CD = _C
    nd = m_per // CD

    @functools.partial(
        pl.kernel,
        out_type=jax.ShapeDtypeStruct((n_work, N * H), jnp.float32),
        mesh=mesh,
        compiler_params=pltpu.CompilerParams(needs_layout_passes=False),
        scratch_types=[
            pltpu.VMEM((2, CD, HP), jnp.float32),
            pltpu.VMEM((2, CD), jnp.int32),
            pltpu.VMEM((N * H,), jnp.float32),
            pltpu.SemaphoreType.DMA((2,)),
        ],
    )
    def sc_denom(w_hbm, seg_hbm, outd_hbm, wbuf, idxbuf, dbuf, sem_in):
        c = lax.axis_index("c")
        s = lax.axis_index("s")
        wid = s * _NC + c
        zero = jnp.zeros((HP,), jnp.float32)
        iota = lax.iota(jnp.int32, HP)
        hmask = iota < H
        base0 = wid * m_per

        def issue_loads(i, b):
            base = pl.multiple_of(base0 + i * CD, 8)
            pltpu.async_copy(w_hbm.at[pl.ds(base, CD)], wbuf.at[b],
                             sem_in.at[b])
            pltpu.async_copy(seg_hbm.at[pl.ds(base, CD)], idxbuf.at[b],
                             sem_in.at[b])

        def wait_loads(b):
            pltpu.make_async_copy(w_hbm.at[pl.ds(0, CD)], wbuf.at[b],
                                  sem_in.at[b]).wait()
            pltpu.make_async_copy(seg_hbm.at[pl.ds(0, CD)], idxbuf.at[b],
                                  sem_in.at[b]).wait()

        issue_loads(0, 0)

        @pl.loop(0, N * H // HP)
        def _(r):
            dbuf[pl.ds(r * HP, HP)] = zero

        def den_accum(i, b):
            wait_loads(b)

            @pl.when(i + 1 < nd)
            def _():
                issue_loads(i + 1, 1 - b)

            @pl.loop(0, CD // HP)
            def _(g):
                iv = idxbuf[b, pl.ds(g * HP, HP)]
                for e16 in range(HP):
                    wv = wbuf[b, g * HP + e16, :]
                    flat = iv[e16] * H + iota
                    plsc.addupdate_scatter(dbuf, [flat], wv, mask=hmask)

        den_accum(0, 0)

        @pl.loop(0, (nd - 1) // 2)
        def _(jj):
            den_accum(2 * jj + 1, 1)
            den_accum(2 * jj + 2, 0)

        pltpu.sync_copy(dbuf, outd_hbm.at[wid])

    parts_d = sc_denom(w_edges, seg)

    NPR = HD // H
    FW = NPR * HD
    NR = N // NPR
    cols = jnp.arange(FW)
    expand = (jnp.arange(HD)[:, None]
              == (cols // HD) * H + (cols % HD) // D).astype(jnp.float32)

    p0v = p0.reshape(NR, FW)
    p1v = p1.reshape(NR, FW)
    den_parts = parts_d.reshape(n_work, DR, HD)

    def _combine_body(p0_ref, p1_ref, dp_ref, ex_ref, o_ref):
        u = p0_ref[...] + p1_ref[...]
        den = jnp.sum(dp_ref[...], axis=0)
        den_rep = jnp.dot(den, ex_ref[...],
                          preferred_element_type=jnp.float32)
        v = u / jnp.where(den_rep > 0, den_rep, 1.0)
        o_ref[...] = jnp.where(v > 0, v, jnp.exp(v) - 1.0)

    out = pl.pallas_call(
        _combine_body,
        grid=(1,),
        in_specs=[
            pl.BlockSpec((NR, FW), lambda i: (0, 0)),
            pl.BlockSpec((NR, FW), lambda i: (0, 0)),
            pl.BlockSpec((n_work, NR, HD), lambda i: (0, 0, 0)),
            pl.BlockSpec((HD, FW), lambda i: (0, 0)),
        ],
        out_specs=pl.BlockSpec((NR, FW), lambda i: (0, 0)),
        out_shape=jax.ShapeDtypeStruct((NR, FW), jnp.float32),
    )(p0v, p1v, den_parts, expand)
    return out.reshape(N, HD)
